# Initial kernel scaffold; baseline (speedup 1.0000x reference)
import jax
import jax.numpy as jnp
from jax.experimental import pallas as pl


def kernel(article_sentences, article_sentences_lengths, attention, num_codes):
    raise NotImplementedError("write your pallas kernel here")


# speedup vs baseline: 1.5682x; 1.5682x over previous
import functools

import jax
import jax.numpy as jnp
from jax import lax
from jax.experimental import pallas as pl
from jax.experimental.pallas import tpu as pltpu

S = 512
L = 16
T = 256


def _col2row(x_col, iota_sub, iota_lane):
    zero = jnp.zeros((), x_col.dtype)
    return jnp.sum(jnp.where(iota_sub == iota_lane, x_col, zero), axis=0,
                   keepdims=True)


def _row2col(x_row, iota_sub, iota_lane):
    zero = jnp.zeros((), x_row.dtype)
    return jnp.sum(jnp.where(iota_sub == iota_lane, x_row, zero), axis=1,
                   keepdims=True)


def _cluster_kernel(num_codes_ref, att_ref, sent_ref, sent_t_ref,
                    len_col_ref, len_row_ref,
                    att_out_ref, sorted_out_ref, group_out_ref, cid_scratch):
    b = pl.program_id(0)
    c = pl.program_id(1)

    iota_sub = lax.broadcasted_iota(jnp.int32, (S, S), 0)
    iota_lane = lax.broadcasted_iota(jnp.int32, (S, S), 1)

    @pl.when(c == 0)
    def _():
        a = sent_ref[0]
        at = sent_t_ref[0]
        acc = None
        for w in range(L // 2):
            p_col = a[:, 2 * w:2 * w + 1] * 32768 + a[:, 2 * w + 1:2 * w + 2]
            p_row = (at[2 * w:2 * w + 1, :] * 32768
                     + at[2 * w + 1:2 * w + 2, :])
            eq_w = p_col == p_row
            acc = eq_w if acc is None else (acc & eq_w)
        cid_scratch[:, :] = jnp.min(jnp.where(acc, iota_lane, S), axis=1,
                                    keepdims=True)

    cid_col = cid_scratch[:, :]

    att_sum = jnp.sum(att_ref[0, 0], axis=1, keepdims=True)
    len_col = len_col_ref[0]
    att_col = jnp.where(len_col == 0, -1.0, att_sum)
    att_row = _col2row(att_col, iota_sub, iota_lane)
    num_sent = jnp.sum(jnp.where(len_row_ref[0] != 0, 1, 0))
    ncodes = num_codes_ref[b]

    gt = att_col > att_row
    tie = (att_col == att_row) & (iota_sub < iota_lane)
    rank_row = jnp.sum((gt | tie).astype(jnp.int32), axis=0, keepdims=True)
    rank_col = _row2col(rank_row, iota_sub, iota_lane)

    onehot = rank_col == iota_lane
    sorted_row = jnp.sum(jnp.where(onehot, iota_sub, 0), axis=0, keepdims=True)
    ordered_row = jnp.sum(jnp.where(onehot, cid_col, 0), axis=0, keepdims=True)
    ordered_col = _row2col(ordered_row, iota_sub, iota_lane)

    eq_ord = ordered_col == ordered_row
    first_row = jnp.min(jnp.where(eq_ord, iota_sub, S), axis=0, keepdims=True)
    first_col = _row2col(first_row, iota_sub, iota_lane)
    is_first_col = first_col == lax.broadcasted_iota(jnp.int32, (S, 1), 0)
    grp_mat = (iota_sub <= first_row) & is_first_col
    group_row = jnp.sum(grp_mat.astype(jnp.int32), axis=0, keepdims=True) - 1

    valid = (lax.broadcasted_iota(jnp.int32, (1, S), 1) < num_sent) & (
        c < ncodes)
    group_row = jnp.where(valid, group_row, -1)

    att_out_ref[0, 0] = att_row
    sorted_out_ref[0, 0] = sorted_row
    group_out_ref[0, 0] = group_row


@jax.jit
def kernel(article_sentences, article_sentences_lengths, attention, num_codes):
    B, S_, L_ = article_sentences.shape
    C = attention.shape[1]
    sent = article_sentences.astype(jnp.int32)
    sent_t = jnp.swapaxes(sent, 1, 2)
    lengths = article_sentences_lengths.astype(jnp.int32)
    len_col = lengths.reshape(B, S_, 1)
    len_row = lengths.reshape(B, 1, S_)

    grid = (B, C)
    out_shapes = (
        jax.ShapeDtypeStruct((B, C, 1, S_), jnp.float32),
        jax.ShapeDtypeStruct((B, C, 1, S_), jnp.int32),
        jax.ShapeDtypeStruct((B, C, 1, S_), jnp.int32),
    )
    out_spec = pl.BlockSpec((1, 1, 1, S_), lambda b, c: (b, c, 0, 0))
    att_s, sorted_s, group_s = pl.pallas_call(
        _cluster_kernel,
        grid=grid,
        in_specs=[
            pl.BlockSpec(memory_space=pltpu.SMEM),
            pl.BlockSpec((1, 1, S_, T), lambda b, c: (b, c, 0, 0)),
            pl.BlockSpec((1, S_, L_), lambda b, c: (b, 0, 0)),
            pl.BlockSpec((1, L_, S_), lambda b, c: (b, 0, 0)),
            pl.BlockSpec((1, S_, 1), lambda b, c: (b, 0, 0)),
            pl.BlockSpec((1, 1, S_), lambda b, c: (b, 0, 0)),
        ],
        out_specs=(out_spec, out_spec, out_spec),
        out_shape=out_shapes,
        scratch_shapes=[pltpu.VMEM((S_, 1), jnp.int32)],
    )(num_codes.astype(jnp.int32), attention, sent, sent_t, len_col, len_row)

    return (att_s.reshape(B, C, S_), sorted_s.reshape(B, C, S_),
            group_s.reshape(B, C, S_))
